# baseline (speedup 1.0000x reference)
import functools

import jax
import jax.numpy as jnp
from jax import lax
from jax.experimental import pallas as pl
from jax.experimental.pallas import tpu as pltpu
from jax.experimental.pallas import tpu_sc as plsc

EMBED = 64
SCALE = float(EMBED) ** 0.5

NC = 2
NS = 16
NW = NC * NS
BB = 128
NBUF = 8

_MESH = dict(core_axis_name="c", subcore_axis_name="s")


def _make_converter(v):
    nfull = v // BB
    rem = v - nfull * BB
    base, extra = divmod(nfull, NW)
    assert rem % 2 == 0

    @functools.partial(
        pl.kernel,
        mesh=plsc.VectorSubcoreMesh(**_MESH),
        out_type=jax.ShapeDtypeStruct((v // 2, 2 * EMBED), jnp.float32),
        scratch_types=[
            pltpu.VMEM((3, EMBED, BB), jnp.float32),
            pltpu.VMEM((2, BB // 2, 2 * EMBED), jnp.float32),
            pltpu.SemaphoreType.DMA,
            pltpu.SemaphoreType.DMA,
        ],
        compiler_params=pltpu.CompilerParams(needs_layout_passes=False),
    )
    def conv(tab_t, rem2, out, panels, pairs, gsem, ssem):
        wid = lax.axis_index("s") * NC + lax.axis_index("c")
        count = base + (wid < extra).astype(jnp.int32)
        iota = lax.iota(jnp.int32, 16)

        @pl.when(wid == (NW - 1) if rem else wid < 0)
        def _():
            pltpu.sync_copy(rem2, pairs.at[0, pl.ds(0, rem // 2)])
            pltpu.sync_copy(pairs.at[0, pl.ds(0, rem // 2)],
                            out.at[pl.ds(nfull * (BB // 2), rem // 2)])

        def blk(i):
            return wid + i * NW

        def transpose_scale(pin, pout):
            pp = panels.at[pin]
            po = pairs.at[pout]

            @plsc.parallel_loop(0, (BB // 16) * EMBED, 1, unroll=8)
            def _(i):
                kk = i >> 6
                g = (i >> 4) & 3
                k = i & 15
                vvec = iota + (kk << 4)
                evec = lax.bitwise_xor(iota, k) + (g << 4)
                val = plsc.load_gather(pp, [evec, vvec])
                plsc.store_scatter(
                    po,
                    [lax.shift_right_logical(vvec, 1),
                     lax.shift_left(lax.bitwise_and(vvec, 1), 6) + evec],
                    val * SCALE)

        def body(i, carry):
            @pl.when(i < count)
            def _():
                pltpu.async_copy(
                    tab_t.at[:, pl.ds(blk(i) * BB, BB)],
                    panels.at[lax.rem(i, 3)], gsem)

            @pl.when(i >= 2)
            def _():
                j = i - 2
                pltpu.make_async_copy(
                    tab_t.at[:, pl.ds(0, BB)],
                    panels.at[lax.rem(j, 3)], gsem).wait()

                @pl.when(j >= 2)
                def _():
                    pltpu.make_async_copy(
                        pairs.at[lax.rem(j, 2)],
                        out.at[pl.ds(0, BB // 2)], ssem).wait()

                transpose_scale(lax.rem(j, 3), lax.rem(j, 2))
                pltpu.async_copy(
                    pairs.at[lax.rem(j, 2)],
                    out.at[pl.ds(blk(j) * (BB // 2), BB // 2)], ssem)
            return carry

        lax.fori_loop(0, count + 2, body, 0)
        pltpu.make_async_copy(
            pairs.at[0], out.at[pl.ds(0, BB // 2)], ssem).wait()
        pltpu.make_async_copy(
            pairs.at[1], out.at[pl.ds(0, BB // 2)], ssem).wait()

    return conv


def _make_lookup(b, s):
    assert b == NW * BB and EMBED == 64

    @functools.partial(
        pl.kernel,
        mesh=plsc.VectorSubcoreMesh(**_MESH),
        out_type=jax.ShapeDtypeStruct((s, 8, NW, 8, BB), jnp.float32),
        scratch_types=[
            pltpu.VMEM((s, BB), jnp.int32),
            pltpu.VMEM((NBUF, BB, EMBED), jnp.float32),
            pltpu.VMEM((2, EMBED, BB), jnp.float32),
            pltpu.SemaphoreType.DMA,
            pltpu.SemaphoreType.DMA,
        ],
        compiler_params=pltpu.CompilerParams(
            use_tc_tiling_on_sc=False, needs_layout_passes=False),
    )
    def k(idx_hbm, table_hbm, out_hbm, idx_v, rows, tiles, gsem, ssem):
        wid = lax.axis_index("s") * NC + lax.axis_index("c")

        pltpu.sync_copy(idx_hbm.at[:, pl.ds(wid * BB, BB)], idx_v)

        iota = lax.iota(jnp.int32, 16)

        def start_gather(step, p):
            pltpu.async_copy(table_hbm.at[idx_v.at[step]], rows.at[p], gsem)

        def wait_gather(p):
            pltpu.make_async_copy(
                table_hbm.at[pl.ds(0, BB)], rows.at[p], gsem).wait()

        def transpose(p, p2):
            rp = rows.at[p]
            tp = tiles.at[p2]

            @plsc.parallel_loop(0, (BB // 16) * EMBED, 1, unroll=8)
            def _(i):
                kk = i >> 6
                g = (i >> 4) & 3
                k = i & 15
                bvec = iota + (kk << 4)
                evec = lax.bitwise_xor(iota, k) + (g << 4)
                val = plsc.load_gather(rp, [bvec, evec])
                plsc.store_scatter(tp, [evec, bvec], val)

        def start_store(step, p2):
            for a in range(8):
                pltpu.async_copy(
                    tiles.at[p2, pl.ds(a * 8, 8)],
                    out_hbm.at[step, a, wid], ssem)

        def wait_store(p2):
            for a in range(8):
                pltpu.make_async_copy(
                    tiles.at[p2, pl.ds(a * 8, 8)],
                    out_hbm.at[0, a, wid], ssem).wait()

        for p in range(NBUF):
            start_gather(p, p)

        def outer(i, carry):
            s0 = i * NBUF
            for p in range(NBUF):
                step = s0 + p
                wait_gather(p)

                @pl.when(step >= 2)
                def _():
                    wait_store(step % 2)

                transpose(p, step % 2)
                start_store(step, step % 2)
                start_gather(jnp.minimum(step + NBUF, s - 1), p)
            return carry

        lax.fori_loop(0, s // NBUF, outer, 0)
        wait_store(0)
        wait_store(1)
        for _ in range(NBUF):
            wait_gather(0)

    return k


def kernel(word_ids, table):
    b, s = word_ids.shape
    v = table.shape[0]
    nfull = v // BB
    rem2 = (SCALE * table[nfull * BB:]).reshape(-1, 2 * EMBED)
    table2 = _make_converter(v)(table.T, rem2)
    out5 = _make_lookup(b, s)(word_ids.T, table2.reshape(v, EMBED))
    return out5.transpose(2, 4, 0, 1, 3).reshape(b, s, EMBED)


# speedup vs baseline: 1.0460x; 1.0460x over previous
import functools

import jax
import jax.numpy as jnp
from jax import lax
from jax.experimental import pallas as pl
from jax.experimental.pallas import tpu as pltpu
from jax.experimental.pallas import tpu_sc as plsc

EMBED = 64
SCALE = float(EMBED) ** 0.5

NC = 2
NS = 16
NW = NC * NS
BB = 128
NBUF = 4

_MESH = dict(core_axis_name="c", subcore_axis_name="s")


def _make_converter(v):
    nfull = v // BB
    rem = v - nfull * BB
    base, extra = divmod(nfull, NW)
    assert rem % 2 == 0

    @functools.partial(
        pl.kernel,
        mesh=plsc.VectorSubcoreMesh(**_MESH),
        out_type=jax.ShapeDtypeStruct((v // 2, 2 * EMBED), jnp.float32),
        scratch_types=[
            pltpu.VMEM((4, EMBED, BB), jnp.float32),
            pltpu.VMEM((2, BB // 2, 2 * EMBED), jnp.float32),
            pltpu.SemaphoreType.DMA,
            pltpu.SemaphoreType.DMA,
        ],
        compiler_params=pltpu.CompilerParams(needs_layout_passes=False),
    )
    def conv(tab_t, rem2, out, panels, pairs, gsem, ssem):
        wid = lax.axis_index("s") * NC + lax.axis_index("c")
        count = base + (wid < extra).astype(jnp.int32)
        iota = lax.iota(jnp.int32, 16)

        @pl.when(wid == (NW - 1) if rem else wid < 0)
        def _():
            pltpu.sync_copy(rem2, pairs.at[0, pl.ds(0, rem // 2)])
            pltpu.sync_copy(pairs.at[0, pl.ds(0, rem // 2)],
                            out.at[pl.ds(nfull * (BB // 2), rem // 2)])

        def blk(i):
            return wid + i * NW

        def transpose_scale(pin, pout):
            pp = panels.at[pin]
            po = pairs.at[pout]

            @plsc.parallel_loop(0, (BB // 16) * EMBED, 1, unroll=8)
            def _(i):
                kk = i >> 6
                g = (i >> 4) & 3
                k = i & 15
                vvec = iota + (kk << 4)
                evec = lax.bitwise_xor(iota, k) + (g << 4)
                val = plsc.load_gather(pp, [evec, vvec])
                plsc.store_scatter(
                    po,
                    [lax.shift_right_logical(vvec, 1),
                     lax.shift_left(lax.bitwise_and(vvec, 1), 6) + evec],
                    val * SCALE)

        def body(i, carry):
            @pl.when(i < count)
            def _():
                pltpu.async_copy(
                    tab_t.at[:, pl.ds(blk(i) * BB, BB)],
                    panels.at[lax.rem(i, 4)], gsem)

            @pl.when(i >= 3)
            def _():
                j = i - 3
                pltpu.make_async_copy(
                    tab_t.at[:, pl.ds(0, BB)],
                    panels.at[lax.rem(j, 4)], gsem).wait()

                @pl.when(j >= 2)
                def _():
                    pltpu.make_async_copy(
                        pairs.at[lax.rem(j, 2)],
                        out.at[pl.ds(0, BB // 2)], ssem).wait()

                transpose_scale(lax.rem(j, 4), lax.rem(j, 2))
                pltpu.async_copy(
                    pairs.at[lax.rem(j, 2)],
                    out.at[pl.ds(blk(j) * (BB // 2), BB // 2)], ssem)
            return carry

        lax.fori_loop(0, count + 3, body, 0)
        pltpu.make_async_copy(
            pairs.at[0], out.at[pl.ds(0, BB // 2)], ssem).wait()
        pltpu.make_async_copy(
            pairs.at[1], out.at[pl.ds(0, BB // 2)], ssem).wait()

    return conv


def _make_lookup(b, s):
    assert b == NW * BB and EMBED == 64

    @functools.partial(
        pl.kernel,
        mesh=plsc.VectorSubcoreMesh(**_MESH),
        out_type=jax.ShapeDtypeStruct((s, 8, NW, 8, BB), jnp.float32),
        scratch_types=[
            pltpu.VMEM((s, BB), jnp.int32),
            pltpu.VMEM((NBUF, BB, EMBED), jnp.float32),
            pltpu.VMEM((2, EMBED, BB), jnp.float32),
            pltpu.SemaphoreType.DMA,
            pltpu.SemaphoreType.DMA,
        ],
        compiler_params=pltpu.CompilerParams(
            use_tc_tiling_on_sc=False, needs_layout_passes=False),
    )
    def k(idx_hbm, table_hbm, out_hbm, idx_v, rows, tiles, gsem, ssem):
        wid = lax.axis_index("s") * NC + lax.axis_index("c")

        pltpu.sync_copy(idx_hbm.at[:, pl.ds(wid * BB, BB)], idx_v)

        iota = lax.iota(jnp.int32, 16)

        def start_gather(step, p):
            pltpu.async_copy(table_hbm.at[idx_v.at[step]], rows.at[p], gsem)

        def wait_gather(p):
            pltpu.make_async_copy(
                table_hbm.at[pl.ds(0, BB)], rows.at[p], gsem).wait()

        def transpose(p, p2):
            rp = rows.at[p]
            tp = tiles.at[p2]

            @plsc.parallel_loop(0, (BB // 16) * EMBED, 1, unroll=8)
            def _(i):
                kk = i >> 6
                g = (i >> 4) & 3
                k = i & 15
                bvec = iota + (kk << 4)
                evec = lax.bitwise_xor(iota, k) + (g << 4)
                val = plsc.load_gather(rp, [bvec, evec])
                plsc.store_scatter(tp, [evec, bvec], val)

        def start_store(step, p2):
            for a in range(8):
                pltpu.async_copy(
                    tiles.at[p2, pl.ds(a * 8, 8)],
                    out_hbm.at[step, a, wid], ssem)

        def wait_store(p2):
            for a in range(8):
                pltpu.make_async_copy(
                    tiles.at[p2, pl.ds(a * 8, 8)],
                    out_hbm.at[0, a, wid], ssem).wait()

        for p in range(NBUF):
            start_gather(p, p)

        def outer(i, carry):
            s0 = i * NBUF
            for p in range(NBUF):
                step = s0 + p
                wait_gather(p)

                @pl.when(step >= 2)
                def _():
                    wait_store(step % 2)

                transpose(p, step % 2)
                start_store(step, step % 2)
                start_gather(jnp.minimum(step + NBUF, s - 1), p)
            return carry

        lax.fori_loop(0, s // NBUF, outer, 0)
        wait_store(0)
        wait_store(1)
        for _ in range(NBUF):
            wait_gather(0)

    return k


def kernel(word_ids, table):
    b, s = word_ids.shape
    v = table.shape[0]
    nfull = v // BB
    rem2 = (SCALE * table[nfull * BB:]).reshape(-1, 2 * EMBED)
    table2 = _make_converter(v)(table.T, rem2)
    out5 = _make_lookup(b, s)(word_ids.T, table2.reshape(v, EMBED))
    return out5.transpose(2, 4, 0, 1, 3).reshape(b, s, EMBED)
